# baseline (speedup 1.0000x reference)
import functools
import math

import jax
import jax.numpy as jnp
from jax import lax
from jax.experimental import pallas as pl
from jax.experimental.pallas import tpu as pltpu
from jax.experimental.pallas import tpu_sc as plsc

N = 10000
E = 320000
D = 128
NC, NS = 2, 16
NW = NC * NS
EPW = E // NW
C = 40
K = 10
GC = K * C
NGROUP = EPW // GC
NP = 10240
RPT = NP // NS
ZROWS = 16
NEG_INV_SQRT_D = float(-1.0 / math.sqrt(D))

BLK = 1000

_HI = jax.lax.Precision.HIGHEST


def _dot(a, b):
    return jnp.dot(a, b, preferred_element_type=jnp.float32, precision=_HI)


def _proj_body(x_ref, w0_ref, w1_ref, wq_ref, wk_ref, wv_ref, wo_ref,
               x1a_ref, xw1_ref, q_ref, k_ref, vo_ref):
    x = x_ref[...]
    x1a_ref[...] = _dot(x, w0_ref[...])
    xw1_ref[...] = _dot(x, w1_ref[...])
    q_ref[...] = _dot(x, wq_ref[...])
    k_ref[...] = _dot(x, wk_ref[...])
    vo_ref[...] = _dot(_dot(x, wv_ref[...]), wo_ref[...])


_proj = pl.pallas_call(
    _proj_body,
    grid=(N // BLK,),
    in_specs=[pl.BlockSpec((BLK, D), lambda i: (i, 0))]
    + [pl.BlockSpec((D, D), lambda i: (0, 0))] * 6,
    out_specs=[pl.BlockSpec((BLK, D), lambda i: (i, 0))] * 5,
    out_shape=[
        jax.ShapeDtypeStruct((N, D), jnp.float32),
        jax.ShapeDtypeStruct((N, D), jnp.float32),
        jax.ShapeDtypeStruct((N, D), jnp.float32),
        jax.ShapeDtypeStruct((N, D), jnp.float32),
        jax.ShapeDtypeStruct((N, D), jnp.float32),
    ],
)


def _comb_body(x1a_ref, p_ref, o_ref):
    o_ref[...] = x1a_ref[...] + p_ref[0] + p_ref[1]


_combine = pl.pallas_call(
    _comb_body,
    grid=(N // BLK,),
    in_specs=[
        pl.BlockSpec((BLK, D), lambda i: (i, 0)),
        pl.BlockSpec((2, BLK, D), lambda i: (0, i, 0)),
    ],
    out_specs=pl.BlockSpec((BLK, D), lambda i: (i, 0)),
    out_shape=jax.ShapeDtypeStruct((N, D), jnp.float32),
)


def _sc_edges_body(src_h, dst_h, s2_h, d2_h, xw1_h, q_h, k_h, vo_h, acc_out,
                   iax, jax_, ibx, jbx, iay, jay, iby, jby,
                   ab0, ab1, qb0, qb1, kb0, kb1, vb0, vb1, zb,
                   acc_sh, sem_ix, sem_iy, sga0, sga1, ssa0, ssa1,
                   sg0, sg1, ss0, ss1):
    cid = lax.axis_index("c")
    sid = lax.axis_index("s")
    wid = sid * NC + cid
    ab = (ab0, ab1)
    qb = (qb0, qb1)
    kb = (kb0, kb1)
    vb = (vb0, vb1)
    sga = (sga0, sga1)
    ssa = (ssa0, ssa1)
    sg = (sg0, sg1)
    ss = (ss0, ss1)
    idx_x = (iax, jax_, ibx, jbx)
    idx_y = (iay, jay, iby, jby)

    def fill_zero_tile():
        def zbody(i, _):
            for j in range(D // 16):
                zb[i, pl.ds(j * 16, 16)] = jnp.zeros((16,), jnp.float32)
            return 0

        lax.fori_loop(0, ZROWS, zbody, 0)

    def zero_acc():
        for r in range(RPT // ZROWS):
            pltpu.sync_copy(zb, acc_sh.at[pl.ds(sid * RPT + r * ZROWS, ZROWS)])

    def flush_acc():
        start = sid * RPT
        pltpu.sync_copy(acc_sh.at[pl.ds(start, RPT)],
                        acc_out.at[pl.ds(cid * NP + start, RPT)])

    def fire_idx(bufs, base, sem):
        for arr, buf in zip((src_h, dst_h, s2_h, d2_h), bufs):
            pltpu.async_copy(arr.at[pl.ds(base, GC)], buf, sem)

    def drain_idx(bufs, base, sem):
        for arr, buf in zip((src_h, dst_h, s2_h, d2_h), bufs):
            pltpu.make_async_copy(arr.at[pl.ds(base, GC)], buf, sem).wait()

    def compute_scores(b):
        def ebody(i, _):
            acc0 = jnp.zeros((16,), jnp.float32)
            acc1 = jnp.zeros((16,), jnp.float32)
            for j in range(D // 32):
                acc0 = acc0 + (qb[b][i, pl.ds(j * 32, 16)]
                               * kb[b][i, pl.ds(j * 32, 16)])
                acc1 = acc1 + (qb[b][i, pl.ds(j * 32 + 16, 16)]
                               * kb[b][i, pl.ds(j * 32 + 16, 16)])
            acc = acc0 + acc1
            lanes = lax.iota(jnp.int32, 16)
            for sh in (8, 4, 2, 1):
                acc = acc + acc.at[lanes ^ sh].get(mode="promise_in_bounds")
            s = acc * NEG_INV_SQRT_D
            for j in range(D // 16):
                vb[b][i, pl.ds(j * 16, 16)] = vb[b][i, pl.ds(j * 16, 16)] * s
            return 0

        lax.fori_loop(0, C, ebody, 0, unroll=2)

    def pipeline(idx):
        ia, ja, ib, jb = idx
        ga, sa, gd, sd = {}, {}, {}, {}

        def fire_a(c, slot):
            return pltpu.async_copy(xw1_h.at[ia.at[pl.ds(c * C, C)]],
                                    ab[slot], sga[slot])

        def fire_kq(c, slot):
            return (
                pltpu.async_copy(k_h.at[ib.at[pl.ds(c * C, C)]],
                                 kb[slot], sg[slot]),
                pltpu.async_copy(q_h.at[jb.at[pl.ds(c * C, C)]],
                                 qb[slot], sg[slot]),
            )

        def fire_vo(c, slot):
            return pltpu.async_copy(vo_h.at[ib.at[pl.ds(c * C, C)]],
                                    vb[slot], sg[slot])

        ga[0] = fire_a(0, 0)
        gd[0] = fire_kq(0, 0) + (fire_vo(0, 0),)
        for c in range(K):
            b = c % 2
            nb = b ^ 1
            if c + 1 < K:
                gkq = fire_kq(c + 1, nb)
            ga[c].wait()
            sa[c] = pltpu.async_copy(ab[b],
                                     acc_sh.at[ja.at[pl.ds(c * C, C)]],
                                     ssa[b], add=True)
            if c + 1 < K:
                if c >= 1:
                    sa[c - 1].wait()
                ga[c + 1] = fire_a(c + 1, nb)
            for cp in gd[c]:
                cp.wait()
            compute_scores(b)
            sd[c] = pltpu.async_copy(vb[b],
                                     acc_sh.at[jb.at[pl.ds(c * C, C)]],
                                     ss[b], add=True)
            if c >= 1:
                sd[c - 1].wait()
            if c + 1 < K:
                gd[c + 1] = gkq + (fire_vo(c + 1, nb),)
        sa[K - 1].wait()
        sd[K - 1].wait()

    fill_zero_tile()
    zero_acc()
    plsc.subcore_barrier()

    base0 = wid * EPW
    fire_idx(idx_x, base0, sem_ix)

    def pair(p, _):
        base_x = wid * EPW + (2 * p) * GC
        base_y = base_x + GC
        drain_idx(idx_x, base_x, sem_ix)
        fire_idx(idx_y, base_y, sem_iy)
        pipeline(idx_x)
        drain_idx(idx_y, base_y, sem_iy)
        fire_idx(idx_x, base_x + 2 * GC, sem_ix)
        pipeline(idx_y)
        return 0

    lax.fori_loop(0, NGROUP // 2, pair, 0)
    base_t = wid * EPW + (NGROUP - 1) * GC
    drain_idx(idx_x, base_t, sem_ix)
    pipeline(idx_x)

    plsc.subcore_barrier()
    flush_acc()


@functools.cache
def _get_sc_edges():
    mesh = plsc.VectorSubcoreMesh(
        core_axis_name="c", subcore_axis_name="s", num_cores=NC, num_subcores=NS
    )
    return pl.kernel(
        _sc_edges_body,
        out_type=jax.ShapeDtypeStruct((NC * NP, D), jnp.float32),
        mesh=mesh,
        scratch_types=[
            pltpu.VMEM((GC,), jnp.int32),
            pltpu.VMEM((GC,), jnp.int32),
            pltpu.VMEM((GC,), jnp.int32),
            pltpu.VMEM((GC,), jnp.int32),
            pltpu.VMEM((GC,), jnp.int32),
            pltpu.VMEM((GC,), jnp.int32),
            pltpu.VMEM((GC,), jnp.int32),
            pltpu.VMEM((GC,), jnp.int32),
            pltpu.VMEM((C, D), jnp.float32),
            pltpu.VMEM((C, D), jnp.float32),
            pltpu.VMEM((C, D), jnp.float32),
            pltpu.VMEM((C, D), jnp.float32),
            pltpu.VMEM((C, D), jnp.float32),
            pltpu.VMEM((C, D), jnp.float32),
            pltpu.VMEM((C, D), jnp.float32),
            pltpu.VMEM((C, D), jnp.float32),
            pltpu.VMEM((ZROWS, D), jnp.float32),
            pltpu.VMEM_SHARED((NP, D), jnp.float32),
        ] + [pltpu.SemaphoreType.DMA] * 10,
    )


def kernel(input, edge_index, edge_index_2, W0, W1, Wq, Wk, Wv, Wo):
    x = input.astype(jnp.float32)
    ei = edge_index.astype(jnp.int32)
    ei2 = edge_index_2.astype(jnp.int32)
    src, dst = ei[0], ei[1]
    s2, d2 = ei2[0], ei2[1]

    x1a, xw1, q, k, vo = _proj(x, W0, W1, Wq, Wk, Wv, Wo)
    p = _get_sc_edges()(src, dst, s2, d2, xw1, q, k, vo)
    p = p.reshape(2, NP, D)[:, :N]
    return _combine(x1a, p)


# speedup vs baseline: 1.6691x; 1.6691x over previous
import functools
import math

import jax
import jax.numpy as jnp
from jax import lax
from jax.experimental import pallas as pl
from jax.experimental.pallas import tpu as pltpu
from jax.experimental.pallas import tpu_sc as plsc

N = 10000
E = 320000
D = 128
NC, NS = 2, 16
NW = NC * NS
EPW = E // NW
C = 40
K = 10
GC = K * C
NGROUP = EPW // GC
NP = 10240
RPT = NP // NS
ZROWS = 16
NEG_INV_SQRT_D = float(-1.0 / math.sqrt(D))

BLK = 1000

_HI = jax.lax.Precision.HIGHEST


def _dot(a, b):
    return jnp.dot(a, b, preferred_element_type=jnp.float32, precision=_HI)


def _proj_body(x_ref, w0_ref, w1_ref, wq_ref, wk_ref, wv_ref, wo_ref,
               x1a_ref, xw1_ref, q_ref, k_ref, vo_ref):
    x = x_ref[...]
    x1a_ref[...] = _dot(x, w0_ref[...])
    xw1_ref[...] = _dot(x, w1_ref[...])
    q_ref[...] = _dot(x, wq_ref[...])
    k_ref[...] = _dot(x, wk_ref[...])
    vo_ref[...] = _dot(_dot(x, wv_ref[...]), wo_ref[...])


_proj = pl.pallas_call(
    _proj_body,
    grid=(N // BLK,),
    in_specs=[pl.BlockSpec((BLK, D), lambda i: (i, 0))]
    + [pl.BlockSpec((D, D), lambda i: (0, 0))] * 6,
    out_specs=[pl.BlockSpec((BLK, D), lambda i: (i, 0))] * 5,
    out_shape=[
        jax.ShapeDtypeStruct((N, D), jnp.float32),
        jax.ShapeDtypeStruct((N, D), jnp.float32),
        jax.ShapeDtypeStruct((N, D), jnp.float32),
        jax.ShapeDtypeStruct((N, D), jnp.float32),
        jax.ShapeDtypeStruct((N, D), jnp.float32),
    ],
)


def _comb_body(x1a_ref, p_ref, o_ref):
    o_ref[...] = x1a_ref[...] + p_ref[0] + p_ref[1]


_combine = pl.pallas_call(
    _comb_body,
    grid=(N // BLK,),
    in_specs=[
        pl.BlockSpec((BLK, D), lambda i: (i, 0)),
        pl.BlockSpec((2, BLK, D), lambda i: (0, i, 0)),
    ],
    out_specs=pl.BlockSpec((BLK, D), lambda i: (i, 0)),
    out_shape=jax.ShapeDtypeStruct((N, D), jnp.float32),
)


def _sc_edges_body(src_h, dst_h, s2_h, d2_h, xw1_h, q_h, k_h, vo_h, acc_out,
                   iax, jax_, ibx, jbx, iay, jay, iby, jby,
                   ab0, ab1, qb0, qb1, kb0, kb1, vb0, vb1, zb,
                   acc_sh, sem_ix, sem_iy, sga0, sga1, ssa0, ssa1,
                   sg0, sg1, ss0, ss1):
    cid = lax.axis_index("c")
    sid = lax.axis_index("s")
    wid = sid * NC + cid
    ab = (ab0, ab1)
    qb = (qb0, qb1)
    kb = (kb0, kb1)
    vb = (vb0, vb1)
    sga = (sga0, sga1)
    ssa = (ssa0, ssa1)
    sg = (sg0, sg1)
    ss = (ss0, ss1)
    idx_x = (iax, jax_, ibx, jbx)
    idx_y = (iay, jay, iby, jby)

    def fill_zero_tile():
        def zbody(i, _):
            for j in range(D // 16):
                zb[i, pl.ds(j * 16, 16)] = jnp.zeros((16,), jnp.float32)
            return 0

        lax.fori_loop(0, ZROWS, zbody, 0)

    def zero_acc():
        for r in range(RPT // ZROWS):
            pltpu.sync_copy(zb, acc_sh.at[pl.ds(sid * RPT + r * ZROWS, ZROWS)])

    def flush_acc():
        start = sid * RPT
        pltpu.sync_copy(acc_sh.at[pl.ds(start, RPT)],
                        acc_out.at[pl.ds(cid * NP + start, RPT)])

    def fire_idx(bufs, base, sem):
        for arr, buf in zip((src_h, dst_h, s2_h, d2_h), bufs):
            pltpu.async_copy(arr.at[pl.ds(base, GC)], buf, sem)

    def drain_idx(bufs, base, sem):
        for arr, buf in zip((src_h, dst_h, s2_h, d2_h), bufs):
            pltpu.make_async_copy(arr.at[pl.ds(base, GC)], buf, sem).wait()

    def compute_scores(b):
        @functools.partial(plsc.parallel_loop, 0, C, unroll=2)
        def ebody(i):
            acc0 = jnp.zeros((16,), jnp.float32)
            acc1 = jnp.zeros((16,), jnp.float32)
            for j in range(D // 32):
                acc0 = acc0 + (qb[b][i, pl.ds(j * 32, 16)]
                               * kb[b][i, pl.ds(j * 32, 16)])
                acc1 = acc1 + (qb[b][i, pl.ds(j * 32 + 16, 16)]
                               * kb[b][i, pl.ds(j * 32 + 16, 16)])
            acc = acc0 + acc1
            lanes = lax.iota(jnp.int32, 16)
            for sh in (8, 4, 2, 1):
                acc = acc + acc.at[lanes ^ sh].get(mode="promise_in_bounds")
            s = acc * NEG_INV_SQRT_D
            for j in range(D // 16):
                vb[b][i, pl.ds(j * 16, 16)] = vb[b][i, pl.ds(j * 16, 16)] * s

    def pipeline(idx):
        ia, ja, ib, jb = idx
        ga, sa, gd, sd = {}, {}, {}, {}

        def fire_a(c, slot):
            return pltpu.async_copy(xw1_h.at[ia.at[pl.ds(c * C, C)]],
                                    ab[slot], sga[slot])

        def fire_kq(c, slot):
            return (
                pltpu.async_copy(k_h.at[ib.at[pl.ds(c * C, C)]],
                                 kb[slot], sg[slot]),
                pltpu.async_copy(q_h.at[jb.at[pl.ds(c * C, C)]],
                                 qb[slot], sg[slot]),
            )

        def fire_vo(c, slot):
            return pltpu.async_copy(vo_h.at[ib.at[pl.ds(c * C, C)]],
                                    vb[slot], sg[slot])

        ga[0] = fire_a(0, 0)
        gd[0] = fire_kq(0, 0) + (fire_vo(0, 0),)
        for c in range(K):
            b = c % 2
            nb = b ^ 1
            if c + 1 < K:
                gkq = fire_kq(c + 1, nb)
            ga[c].wait()
            sa[c] = pltpu.async_copy(ab[b],
                                     acc_sh.at[ja.at[pl.ds(c * C, C)]],
                                     ssa[b], add=True)
            if c + 1 < K:
                if c >= 1:
                    sa[c - 1].wait()
                ga[c + 1] = fire_a(c + 1, nb)
            for cp in gd[c]:
                cp.wait()
            compute_scores(b)
            sd[c] = pltpu.async_copy(vb[b],
                                     acc_sh.at[jb.at[pl.ds(c * C, C)]],
                                     ss[b], add=True)
            if c >= 1:
                sd[c - 1].wait()
            if c + 1 < K:
                gd[c + 1] = gkq + (fire_vo(c + 1, nb),)
        sa[K - 1].wait()
        sd[K - 1].wait()

    fill_zero_tile()
    zero_acc()
    plsc.subcore_barrier()

    base0 = wid * EPW
    fire_idx(idx_x, base0, sem_ix)

    def pair(p, _):
        base_x = wid * EPW + (2 * p) * GC
        base_y = base_x + GC
        drain_idx(idx_x, base_x, sem_ix)
        fire_idx(idx_y, base_y, sem_iy)
        pipeline(idx_x)
        drain_idx(idx_y, base_y, sem_iy)
        fire_idx(idx_x, base_x + 2 * GC, sem_ix)
        pipeline(idx_y)
        return 0

    lax.fori_loop(0, NGROUP // 2, pair, 0)
    base_t = wid * EPW + (NGROUP - 1) * GC
    drain_idx(idx_x, base_t, sem_ix)
    pipeline(idx_x)

    plsc.subcore_barrier()
    flush_acc()


@functools.cache
def _get_sc_edges():
    mesh = plsc.VectorSubcoreMesh(
        core_axis_name="c", subcore_axis_name="s", num_cores=NC, num_subcores=NS
    )
    return pl.kernel(
        _sc_edges_body,
        out_type=jax.ShapeDtypeStruct((NC * NP, D), jnp.float32),
        mesh=mesh,
        scratch_types=[
            pltpu.VMEM((GC,), jnp.int32),
            pltpu.VMEM((GC,), jnp.int32),
            pltpu.VMEM((GC,), jnp.int32),
            pltpu.VMEM((GC,), jnp.int32),
            pltpu.VMEM((GC,), jnp.int32),
            pltpu.VMEM((GC,), jnp.int32),
            pltpu.VMEM((GC,), jnp.int32),
            pltpu.VMEM((GC,), jnp.int32),
            pltpu.VMEM((C, D), jnp.float32),
            pltpu.VMEM((C, D), jnp.float32),
            pltpu.VMEM((C, D), jnp.float32),
            pltpu.VMEM((C, D), jnp.float32),
            pltpu.VMEM((C, D), jnp.float32),
            pltpu.VMEM((C, D), jnp.float32),
            pltpu.VMEM((C, D), jnp.float32),
            pltpu.VMEM((C, D), jnp.float32),
            pltpu.VMEM((ZROWS, D), jnp.float32),
            pltpu.VMEM_SHARED((NP, D), jnp.float32),
        ] + [pltpu.SemaphoreType.DMA] * 10,
    )


def kernel(input, edge_index, edge_index_2, W0, W1, Wq, Wk, Wv, Wo):
    x = input.astype(jnp.float32)
    ei = edge_index.astype(jnp.int32)
    ei2 = edge_index_2.astype(jnp.int32)
    src, dst = ei[0], ei[1]
    s2, d2 = ei2[0], ei2[1]

    x1a, xw1, q, k, vo = _proj(x, W0, W1, Wq, Wk, Wv, Wo)
    p = _get_sc_edges()(src, dst, s2, d2, xw1, q, k, vo)
    p = p.reshape(2, NP, D)[:, :N]
    return _combine(x1a, p)
